# baseline (speedup 1.0000x reference)
import jax
import jax.numpy as jnp
from jax.experimental import pallas as pl

BM = 4096
BN = 2048
LANES = 128


def _b2_kernel(b_ref, out_ref):
    b = b_ref[...]
    nch = out_ref.shape[0]
    out_ref[...] = jnp.sum((b * b).reshape(nch, LANES, b.shape[1]), axis=2)


def _nn_kernel(a_ref, b_ref, b2_ref, dist_ref, idx_ref):
    a = a_ref[...]
    b2 = b2_ref[...]
    n = b_ref.shape[0]
    nch = BN // LANES

    m = jnp.full((BM, LANES), jnp.inf, jnp.float32)
    kk = jnp.zeros((BM, LANES), jnp.int32)
    for j in range(n // BN):
        x = jax.lax.dot_general(
            a * -2.0, b_ref[j * BN:(j + 1) * BN, :],
            (((1,), (1,)), ((), ())),
            preferred_element_type=jnp.float32, precision=jax.lax.Precision.HIGHEST,
        )
        for t in range(nch):
            g = j * nch + t
            c = x[:, t * LANES:(t + 1) * LANES] + b2[g:g + 1, :]
            better = c < m
            kk = jnp.where(better, g, kk)
            m = jnp.minimum(c, m)

    lane_arg = jnp.argmin(m, axis=1).astype(jnp.int32)
    row_min = jnp.min(m, axis=1)
    onehot = (jax.lax.broadcasted_iota(jnp.int32, (BM, LANES), 1)
              == lane_arg[:, None])
    chunk = jnp.max(jnp.where(onehot, kk, 0), axis=1)
    a2 = jnp.sum(a * a, axis=1)
    dist_ref[...] = jnp.sqrt(jnp.maximum(row_min + a2, 0.0))[:, None]
    idx_ref[...] = (chunk * LANES + lane_arg)[:, None]


def kernel(desc1, desc2):
    m, k = desc1.shape
    n, _ = desc2.shape
    m_blocks = m // BM

    b2 = pl.pallas_call(
        _b2_kernel,
        out_shape=jax.ShapeDtypeStruct((n // LANES, LANES), jnp.float32),
    )(desc2)

    dists, idxs = pl.pallas_call(
        _nn_kernel,
        grid=(m_blocks,),
        in_specs=[
            pl.BlockSpec((BM, k), lambda i: (i, 0)),
            pl.BlockSpec((n, k), lambda i: (0, 0)),
            pl.BlockSpec((n // LANES, LANES), lambda i: (0, 0)),
        ],
        out_specs=[
            pl.BlockSpec((BM, 1), lambda i: (i, 0)),
            pl.BlockSpec((BM, 1), lambda i: (i, 0)),
        ],
        out_shape=[
            jax.ShapeDtypeStruct((m, 1), jnp.float32),
            jax.ShapeDtypeStruct((m, 1), jnp.int32),
        ],
    )(desc1, desc2, b2)

    idxs_in_1 = jnp.arange(m, dtype=jnp.int32).reshape(-1, 1)
    matches_idxs = jnp.concatenate([idxs_in_1, idxs], axis=1)
    return (dists, matches_idxs)


# speedup vs baseline: 5.0082x; 5.0082x over previous
import jax
import jax.numpy as jnp
from jax.experimental import pallas as pl

BM = 4096
BN = 2048
LANES = 128


def _b2_kernel(b_ref, out_ref):
    b = b_ref[...]
    nch = out_ref.shape[0]
    out_ref[...] = jnp.sum((b * b).reshape(nch, LANES, b.shape[1]), axis=2)


def _nn_kernel(a_ref, b_ref, b2_ref, dist_ref, idx_ref):
    a = a_ref[...]
    b2 = b2_ref[...]
    n = b_ref.shape[0]
    nch = BN // LANES

    m = jnp.full((BM, LANES), jnp.inf, jnp.float32)
    kk = jnp.zeros((BM, LANES), jnp.int32)
    for j in range(n // BN):
        x = jax.lax.dot_general(
            a * -2.0, b_ref[j * BN:(j + 1) * BN, :],
            (((1,), (1,)), ((), ())),
            preferred_element_type=jnp.float32,
        )
        for t in range(nch):
            g = j * nch + t
            c = x[:, t * LANES:(t + 1) * LANES] + b2[g:g + 1, :]
            better = c < m
            kk = jnp.where(better, g, kk)
            m = jnp.minimum(c, m)

    lane_arg = jnp.argmin(m, axis=1).astype(jnp.int32)
    row_min = jnp.min(m, axis=1)
    onehot = (jax.lax.broadcasted_iota(jnp.int32, (BM, LANES), 1)
              == lane_arg[:, None])
    chunk = jnp.max(jnp.where(onehot, kk, 0), axis=1)
    a2 = jnp.sum(a * a, axis=1)
    dist_ref[...] = jnp.sqrt(jnp.maximum(row_min + a2, 0.0))[:, None]
    idx_ref[...] = (chunk * LANES + lane_arg)[:, None]


def kernel(desc1, desc2):
    m, k = desc1.shape
    n, _ = desc2.shape
    m_blocks = m // BM

    b2 = pl.pallas_call(
        _b2_kernel,
        out_shape=jax.ShapeDtypeStruct((n // LANES, LANES), jnp.float32),
    )(desc2)

    dists, idxs = pl.pallas_call(
        _nn_kernel,
        grid=(m_blocks,),
        in_specs=[
            pl.BlockSpec((BM, k), lambda i: (i, 0)),
            pl.BlockSpec((n, k), lambda i: (0, 0)),
            pl.BlockSpec((n // LANES, LANES), lambda i: (0, 0)),
        ],
        out_specs=[
            pl.BlockSpec((BM, 1), lambda i: (i, 0)),
            pl.BlockSpec((BM, 1), lambda i: (i, 0)),
        ],
        out_shape=[
            jax.ShapeDtypeStruct((m, 1), jnp.float32),
            jax.ShapeDtypeStruct((m, 1), jnp.int32),
        ],
    )(desc1, desc2, b2)

    idxs_in_1 = jnp.arange(m, dtype=jnp.int32).reshape(-1, 1)
    matches_idxs = jnp.concatenate([idxs_in_1, idxs], axis=1)
    return (dists, matches_idxs)


# speedup vs baseline: 5.0218x; 1.0027x over previous
import jax
import jax.numpy as jnp
from jax.experimental import pallas as pl

BM = 4096
BN = 1024
LANES = 128


def _b2_kernel(b_ref, out_ref):
    b = b_ref[...]
    nch = out_ref.shape[0]
    out_ref[...] = jnp.sum((b * b).reshape(nch, LANES, b.shape[1]), axis=2)


def _nn_kernel(a_ref, b_ref, b2_ref, dist_ref, idx_ref):
    a = a_ref[...]
    b2 = b2_ref[...]
    n = b_ref.shape[0]
    nch = BN // LANES

    m = jnp.full((BM, LANES), jnp.inf, jnp.float32)
    kk = jnp.zeros((BM, LANES), jnp.int32)
    for j in range(n // BN):
        x = jax.lax.dot_general(
            a * -2.0, b_ref[j * BN:(j + 1) * BN, :],
            (((1,), (1,)), ((), ())),
            preferred_element_type=jnp.float32,
        )
        for t in range(nch):
            g = j * nch + t
            c = x[:, t * LANES:(t + 1) * LANES] + b2[g:g + 1, :]
            better = c < m
            kk = jnp.where(better, g, kk)
            m = jnp.minimum(c, m)

    lane_arg = jnp.argmin(m, axis=1).astype(jnp.int32)
    row_min = jnp.min(m, axis=1)
    onehot = (jax.lax.broadcasted_iota(jnp.int32, (BM, LANES), 1)
              == lane_arg[:, None])
    chunk = jnp.max(jnp.where(onehot, kk, 0), axis=1)
    a2 = jnp.sum(a * a, axis=1)
    dist_ref[...] = jnp.sqrt(jnp.maximum(row_min + a2, 0.0))[:, None]
    idx_ref[...] = (chunk * LANES + lane_arg)[:, None]


def kernel(desc1, desc2):
    m, k = desc1.shape
    n, _ = desc2.shape
    m_blocks = m // BM

    b2 = pl.pallas_call(
        _b2_kernel,
        out_shape=jax.ShapeDtypeStruct((n // LANES, LANES), jnp.float32),
    )(desc2)

    dists, idxs = pl.pallas_call(
        _nn_kernel,
        grid=(m_blocks,),
        in_specs=[
            pl.BlockSpec((BM, k), lambda i: (i, 0)),
            pl.BlockSpec((n, k), lambda i: (0, 0)),
            pl.BlockSpec((n // LANES, LANES), lambda i: (0, 0)),
        ],
        out_specs=[
            pl.BlockSpec((BM, 1), lambda i: (i, 0)),
            pl.BlockSpec((BM, 1), lambda i: (i, 0)),
        ],
        out_shape=[
            jax.ShapeDtypeStruct((m, 1), jnp.float32),
            jax.ShapeDtypeStruct((m, 1), jnp.int32),
        ],
    )(desc1, desc2, b2)

    idxs_in_1 = jnp.arange(m, dtype=jnp.int32).reshape(-1, 1)
    matches_idxs = jnp.concatenate([idxs_in_1, idxs], axis=1)
    return (dists, matches_idxs)
